# baseline (speedup 1.0000x reference)
import functools

import jax
import jax.numpy as jnp
from jax import lax
from jax.experimental import pallas as pl
from jax.experimental.pallas import tpu as pltpu
from jax.experimental.pallas import tpu_sc as plsc

_NC = 2
_NS = 16
_NW = _NC * _NS
_L = 16
_BB = 128
_H = 200
_D = 32
_CS = 4
_HW = 50
_TP = 129


@jax.jit
def _sc_gather(xv, table):
  mesh = plsc.VectorSubcoreMesh(
      core_axis_name="c", subcore_axis_name="s",
      num_cores=_NC, num_subcores=_NS)

  @functools.partial(
      pl.kernel,
      out_type=jax.ShapeDtypeStruct((_H, _D // 8, _NW, 8, _BB), jnp.float32),
      mesh=mesh,
      compiler_params=pltpu.CompilerParams(use_tc_tiling_on_sc=False,
                                           needs_layout_passes=False),
      scratch_types=[
          pltpu.VMEM((7, _CS, 8, _BB), jnp.int32),
          pltpu.VMEM((3, _CS, _BB, _D), jnp.float32),
          pltpu.VMEM((2, _CS, _D, _TP), jnp.float32),
          pltpu.SemaphoreType.DMA,
          pltpu.SemaphoreType.DMA,
          pltpu.SemaphoreType.DMA,
          pltpu.SemaphoreType.DMA,
          pltpu.SemaphoreType.DMA,
      ],
  )
  def body(xv_hbm, table_hbm, out_hbm, idx_v, rows_v, t_v,
           g0, g1, g2, w0, w1):
    wid = lax.axis_index("s") * _NC + lax.axis_index("c")
    ct4 = wid % 8
    h0 = (wid // 8) * _HW
    kb = h0 // 8
    gsem = (g0, g1, g2)
    wsem = (w0, w1)
    iota = lax.iota(jnp.int32, _L)
    rids = [iota + (g * _L) for g in range(8)]

    for k in range(7):
      pltpu.sync_copy(xv_hbm.at[kb + k, pl.ds(ct4 * _CS, _CS)], idx_v.at[k])

    def idx_row(h, cs):
      return idx_v.at[h // 8 - kb, cs, h % 8]

    def start_gathers(h, sl):
      for cs in range(_CS):
        pltpu.async_copy(table_hbm.at[idx_row(h, cs)], rows_v.at[sl, cs],
                         gsem[sl])

    def wait_gathers(sl):
      for cs in range(_CS):
        pltpu.make_async_copy(table_hbm.at[idx_v.at[0, 0, 0]],
                              rows_v.at[sl, cs], gsem[sl]).wait()

    def start_writes(h, sl):
      for cs in range(_CS):
        for d8 in range(_D // 8):
          pltpu.async_copy(t_v.at[sl, cs, pl.ds(8 * d8, 8), pl.ds(0, _BB)],
                           out_hbm.at[h, d8, ct4 * _CS + cs], wsem[sl])

    def wait_writes(h, sl):
      for cs in range(_CS):
        for d8 in range(_D // 8):
          pltpu.make_async_copy(
              t_v.at[sl, cs, pl.ds(8 * d8, 8), pl.ds(0, _BB)],
              out_hbm.at[h, d8, ct4 * _CS + cs], wsem[sl]).wait()

    ones = jnp.full((_L,), 1, jnp.int32)

    def transpose_block(rs, ts):
      def csloop(cs, _):
        block = rows_v.at[rs, cs]
        tdst = t_v.at[ts, cs]
        bv = jnp.full((_L,), 0, jnp.int32)
        for b in range(_BB):
          v0 = block[b, pl.ds(0, _L)]
          v1 = block[b, pl.ds(_L, _L)]
          plsc.store_scatter(tdst, [iota, bv], v0)
          plsc.store_scatter(tdst, [iota + _L, bv], v1)
          bv = bv + ones
        return ()

      lax.fori_loop(0, _CS, csloop, ())

    for j in range(3):
      start_gathers(h0 + j, j)

    def six(q, _):
      for j in range(6):
        u = 6 * q + j
        h = h0 + u
        rs = j % 3
        ts = j % 2
        wait_gathers(rs)

        @pl.when(u >= 2)
        def _():
          wait_writes(h, ts)

        transpose_block(rs, ts)
        start_writes(h, ts)

        @pl.when(u + 3 < _HW)
        def _():
          start_gathers(h + 3, rs)
      return ()

    lax.fori_loop(0, (_HW - 2) // 6, six, (), unroll=False)
    for u in range(_HW - 2, _HW):
      h = h0 + u
      rs = u % 3
      ts = u % 2
      wait_gathers(rs)
      wait_writes(h, ts)
      transpose_block(rs, ts)
      start_writes(h, ts)
    for u in range(_HW - 2, _HW):
      wait_writes(h0 + u, u % 2)

  return body(xv, table)


def kernel(table, x):
  xv = (x.astype(jnp.int32)
        .reshape(32, 128, 25, 8)
        .transpose(2, 0, 3, 1))
  out5 = _sc_gather(xv, table)
  return out5.transpose(2, 4, 0, 1, 3).reshape(4096, _H, _D)


# speedup vs baseline: 1.0433x; 1.0433x over previous
import functools

import jax
import jax.numpy as jnp
from jax import lax
from jax.experimental import pallas as pl
from jax.experimental.pallas import tpu as pltpu
from jax.experimental.pallas import tpu_sc as plsc

_NC = 2
_NS = 16
_NW = _NC * _NS
_L = 16
_BB = 128
_H = 200
_D = 32
_CS = 4
_HW = 50
_TP = 129


@jax.jit
def _sc_gather(xv, table):
  mesh = plsc.VectorSubcoreMesh(
      core_axis_name="c", subcore_axis_name="s",
      num_cores=_NC, num_subcores=_NS)

  @functools.partial(
      pl.kernel,
      out_type=jax.ShapeDtypeStruct((_H, _D // 8, _NW, 8, _BB), jnp.float32),
      mesh=mesh,
      compiler_params=pltpu.CompilerParams(use_tc_tiling_on_sc=False,
                                           needs_layout_passes=False),
      scratch_types=[
          pltpu.VMEM((7, _CS, 8, _BB), jnp.int32),
          pltpu.VMEM((2, _CS, _BB, _D), jnp.float32),
          pltpu.VMEM((2, _CS, _D, _TP), jnp.float32),
          pltpu.SemaphoreType.DMA,
          pltpu.SemaphoreType.DMA,
          pltpu.SemaphoreType.DMA,
          pltpu.SemaphoreType.DMA,
      ],
  )
  def body(xv_hbm, table_hbm, out_hbm, idx_v, rows_v, t_v, g0, g1, w0, w1):
    wid = lax.axis_index("s") * _NC + lax.axis_index("c")
    ct4 = wid % 8
    h0 = (wid // 8) * _HW
    kb = h0 // 8
    gsem = (g0, g1)
    wsem = (w0, w1)
    iota = lax.iota(jnp.int32, _L)
    rids = [iota + (g * _L) for g in range(8)]

    for k in range(7):
      pltpu.sync_copy(xv_hbm.at[kb + k, pl.ds(ct4 * _CS, _CS)], idx_v.at[k])

    def idx_row(h, cs):
      return idx_v.at[h // 8 - kb, cs, h % 8]

    def start_gathers(h, sl):
      for cs in range(_CS):
        pltpu.async_copy(table_hbm.at[idx_row(h, cs)], rows_v.at[sl, cs],
                         gsem[sl])

    def wait_gathers(sl):
      for cs in range(_CS):
        pltpu.make_async_copy(table_hbm.at[idx_v.at[0, 0, 0]],
                              rows_v.at[sl, cs], gsem[sl]).wait()

    def start_writes(h, sl):
      for cs in range(_CS):
        for d8 in range(_D // 8):
          pltpu.async_copy(t_v.at[sl, cs, pl.ds(8 * d8, 8), pl.ds(0, _BB)],
                           out_hbm.at[h, d8, ct4 * _CS + cs], wsem[sl])

    def wait_writes(h, sl):
      for cs in range(_CS):
        for d8 in range(_D // 8):
          pltpu.make_async_copy(
              t_v.at[sl, cs, pl.ds(8 * d8, 8), pl.ds(0, _BB)],
              out_hbm.at[h, d8, ct4 * _CS + cs], wsem[sl]).wait()

    ones = jnp.full((_L,), 1, jnp.int32)

    def transpose_block(sl):
      def csloop(cs, _):
        block = rows_v.at[sl, cs]
        tdst = t_v.at[sl, cs]
        bv = jnp.full((_L,), 0, jnp.int32)
        for b in range(_BB):
          v0 = block[b, pl.ds(0, _L)]
          v1 = block[b, pl.ds(_L, _L)]
          plsc.store_scatter(tdst, [iota, bv], v0)
          plsc.store_scatter(tdst, [iota + _L, bv], v1)
          bv = bv + ones
        return ()

      lax.fori_loop(0, _CS, csloop, ())

    start_gathers(h0, 0)
    start_gathers(h0 + 1, 1)

    def pair(p, _):
      for sl in range(2):
        h = h0 + 2 * p + sl
        wait_gathers(sl)

        @pl.when(p >= 1)
        def _():
          wait_writes(h, sl)

        transpose_block(sl)
        start_writes(h, sl)

        @pl.when(p < _HW // 2 - 1)
        def _():
          start_gathers(h + 2, sl)
      return ()

    lax.fori_loop(0, _HW // 2, pair, (), unroll=False)
    for sl in range(2):
      wait_writes(h0 + _HW - 2 + sl, sl)

  return body(xv, table)


def kernel(table, x):
  xv = (x.astype(jnp.int32)
        .reshape(32, 128, 25, 8)
        .transpose(2, 0, 3, 1))
  out5 = _sc_gather(xv, table)
  return out5.transpose(2, 4, 0, 1, 3).reshape(4096, _H, _D)
